# initial kernel scaffold (unmeasured)
import jax
import jax.numpy as jnp
from jax import lax
from jax.experimental import pallas as pl
from jax.experimental.pallas import tpu as pltpu

N_DEV = 4
B = 2
S = 512
H = 8
Dh = 64
D_MODEL = 768
BLK = 64


def kernel(x, Wq, K_ext, V_ext, Wo):
    def body(x_ref, wq_ref, k_ref, v_ref, wo_ref, out_ref,
             comm_ref, send_sems, recv_sems):
        my = lax.axis_index("i")
        left = lax.rem(my + N_DEV - 1, N_DEV)
        right = lax.rem(my + 1, N_DEV)

        barrier = pltpu.get_barrier_semaphore()
        for nbr in (left, right):
            pl.semaphore_signal(barrier, inc=1, device_id=(nbr,),
                                device_id_type=pl.DeviceIdType.MESH)
        pl.semaphore_wait(barrier, 2)

        comm_ref[0, 0] = k_ref[...].astype(jnp.bfloat16)
        comm_ref[0, 1] = v_ref[...].astype(jnp.bfloat16)

        xq = x_ref[...].reshape(B * S, D_MODEL).astype(jnp.bfloat16)
        wq = wq_ref[...].astype(jnp.bfloat16)
        q = jnp.dot(xq, wq, preferred_element_type=jnp.float32)
        q = (q * 0.125).astype(jnp.bfloat16)

        rows = lax.broadcasted_iota(jnp.int32, (S, S), 0)
        cols = lax.broadcasted_iota(jnp.int32, (S, S), 1)
        qb = my * (S // BLK) + rows // BLK

        acc = [[None] * H for _ in range(B)]
        den = [[None] * H for _ in range(B)]
        rdmas = []
        for h in range(N_DEV):
            if h > 0:
                rdmas[h - 1].wait()
            if h < N_DEV - 1:
                r = pltpu.make_async_remote_copy(
                    src_ref=comm_ref.at[h],
                    dst_ref=comm_ref.at[h + 1],
                    send_sem=send_sems.at[h],
                    recv_sem=recv_sems.at[h],
                    device_id=(right,),
                    device_id_type=pl.DeviceIdType.MESH,
                )
                r.start()
                rdmas.append(r)

            origin = lax.rem(my - h + N_DEV, N_DEV)
            kb = origin * (S // BLK) + cols // BLK
            mask = (qb == kb) | (kb == 0) | (lax.rem(qb + kb, 3) == 0)
            for b in range(B):
                for hd in range(H):
                    qbh = q[b * S:(b + 1) * S, hd * Dh:(hd + 1) * Dh]
                    kbh = comm_ref[h, 0, b, :, hd, :]
                    s = lax.dot_general(
                        qbh, kbh, (((1,), (1,)), ((), ())),
                        preferred_element_type=jnp.float32)
                    w = jnp.where(mask, jnp.exp(s), 0.0)
                    vbh = comm_ref[h, 1, b, :, hd, :]
                    pv = jnp.dot(w.astype(jnp.bfloat16), vbh,
                                 preferred_element_type=jnp.float32)
                    dsum = jnp.sum(w, axis=1, keepdims=True)
                    if h == 0:
                        acc[b][hd] = pv
                        den[b][hd] = dsum
                    else:
                        acc[b][hd] = acc[b][hd] + pv
                        den[b][hd] = den[b][hd] + dsum

        ctx = jnp.concatenate(
            [jnp.concatenate([acc[b][hd] / den[b][hd] for hd in range(H)],
                             axis=1)
             for b in range(B)], axis=0)
        out = jnp.dot(ctx.astype(jnp.bfloat16),
                      wo_ref[...].astype(jnp.bfloat16),
                      preferred_element_type=jnp.float32)
        out_ref[...] = out.reshape(B, S, D_MODEL)

    return pl.pallas_call(
        body,
        out_shape=jax.ShapeDtypeStruct((B, S, D_MODEL), jnp.float32),
        in_specs=[pl.BlockSpec(memory_space=pltpu.VMEM)] * 5,
        out_specs=pl.BlockSpec(memory_space=pltpu.VMEM),
        scratch_shapes=[
            pltpu.VMEM((N_DEV, 2, B, S, H, Dh), jnp.bfloat16),
            pltpu.SemaphoreType.DMA((N_DEV - 1,)),
            pltpu.SemaphoreType.DMA((N_DEV - 1,)),
        ],
        compiler_params=pltpu.CompilerParams(collective_id=0),
    )(x, Wq, K_ext, V_ext, Wo)


# baseline (device time: 102923 ns/iter reference)
import jax
import jax.numpy as jnp
from jax import lax
from jax.experimental import pallas as pl
from jax.experimental.pallas import tpu as pltpu

N_DEV = 4
B = 2
S = 512
H = 8
Dh = 64
D_MODEL = 768
BLK = 64


def kernel(x, Wq, K_ext, V_ext, Wo):
    def body(x_ref, wq_ref, k_ref, v_ref, wo_ref, out_ref,
             comm_ref, acc_ref, ctx_ref, send_sems, recv_sems):
        my = lax.axis_index("i")
        left = lax.rem(my + N_DEV - 1, N_DEV)
        right = lax.rem(my + 1, N_DEV)

        barrier = pltpu.get_barrier_semaphore()
        for nbr in (left, right):
            pl.semaphore_signal(barrier, inc=1, device_id=(nbr,),
                                device_id_type=pl.DeviceIdType.MESH)
        pl.semaphore_wait(barrier, 2)

        comm_ref[0, 0] = k_ref[...].reshape(B, S, H * Dh).astype(jnp.bfloat16)
        comm_ref[0, 1] = v_ref[...].reshape(B, S, H * Dh).astype(jnp.bfloat16)

        xq = x_ref[...].reshape(B * S, D_MODEL).astype(jnp.bfloat16)
        wq = wq_ref[...].astype(jnp.bfloat16)
        q = jnp.dot(xq, wq, preferred_element_type=jnp.float32)
        q = (q * 0.125).astype(jnp.bfloat16)

        rows = lax.broadcasted_iota(jnp.int32, (S, S), 0)
        cols = lax.broadcasted_iota(jnp.int32, (S, S), 1)
        qb = my * (S // BLK) + rows // BLK

        den = [[None] * H for _ in range(B)]
        rdmas = []
        for h in range(N_DEV):
            if h > 0:
                rdmas[h - 1].wait()
            if h < N_DEV - 1:
                r = pltpu.make_async_remote_copy(
                    src_ref=comm_ref.at[h],
                    dst_ref=comm_ref.at[h + 1],
                    send_sem=send_sems.at[h],
                    recv_sem=recv_sems.at[h],
                    device_id=(right,),
                    device_id_type=pl.DeviceIdType.MESH,
                )
                r.start()
                rdmas.append(r)

            origin = lax.rem(my - h + N_DEV, N_DEV)
            kb = origin * (S // BLK) + cols // BLK
            mask = (qb == kb) | (kb == 0) | (lax.rem(qb + kb, 3) == 0)
            for b in range(B):
                for hd in range(H):
                    qbh = q[b * S:(b + 1) * S, hd * Dh:(hd + 1) * Dh]
                    kbh = comm_ref[h, 0, b, :, hd * Dh:(hd + 1) * Dh]
                    s = lax.dot_general(
                        qbh, kbh, (((1,), (1,)), ((), ())),
                        preferred_element_type=jnp.float32)
                    w = jnp.where(mask, jnp.exp(s), 0.0)
                    vbh = comm_ref[h, 1, b, :, hd * Dh:(hd + 1) * Dh]
                    pv = jnp.dot(w.astype(jnp.bfloat16), vbh,
                                 preferred_element_type=jnp.float32)
                    dsum = jnp.sum(w, axis=1, keepdims=True)
                    if h == 0:
                        acc_ref[pl.ds(b * S, S), pl.ds(hd * Dh, Dh)] = pv
                        den[b][hd] = dsum
                    else:
                        acc_ref[pl.ds(b * S, S), pl.ds(hd * Dh, Dh)] += pv
                        den[b][hd] = den[b][hd] + dsum

        for b in range(B):
            for hd in range(H):
                piece = acc_ref[pl.ds(b * S, S), pl.ds(hd * Dh, Dh)]
                ctx_ref[pl.ds(b * S, S), pl.ds(hd * Dh, Dh)] = (
                    piece / den[b][hd]).astype(jnp.bfloat16)

        out = jnp.dot(ctx_ref[...], wo_ref[...].astype(jnp.bfloat16),
                      preferred_element_type=jnp.float32)
        out_ref[...] = out.reshape(B, S, D_MODEL)

    return pl.pallas_call(
        body,
        out_shape=jax.ShapeDtypeStruct((B, S, D_MODEL), jnp.float32),
        in_specs=[pl.BlockSpec(memory_space=pltpu.VMEM)] * 5,
        out_specs=pl.BlockSpec(memory_space=pltpu.VMEM),
        scratch_shapes=[
            pltpu.VMEM((N_DEV, 2, B, S, H * Dh), jnp.bfloat16),
            pltpu.VMEM((B * S, H * Dh), jnp.float32),
            pltpu.VMEM((B * S, H * Dh), jnp.bfloat16),
            pltpu.SemaphoreType.DMA((N_DEV - 1,)),
            pltpu.SemaphoreType.DMA((N_DEV - 1,)),
        ],
        compiler_params=pltpu.CompilerParams(
            collective_id=0, vmem_limit_bytes=60 * 1024 * 1024),
    )(x, Wq, K_ext, V_ext, Wo)


# device time: 68642 ns/iter; 1.4994x vs baseline; 1.4994x over previous
import jax
import jax.numpy as jnp
from jax import lax
from jax.experimental import pallas as pl
from jax.experimental.pallas import tpu as pltpu

N_DEV = 4
B = 2
S = 512
H = 8
Dh = 64
D_MODEL = 768
BLK = 64


def kernel(x, Wq, K_ext, V_ext, Wo):
    def body(x_ref, wq_ref, k_ref, v_ref, wo_ref, out_ref,
             comm_ref, acc_ref, ctx_ref,
             send_sems, recv_sems):
        my = lax.axis_index("i")
        left = lax.rem(my + N_DEV - 1, N_DEV)
        right = lax.rem(my + 1, N_DEV)

        barrier = pltpu.get_barrier_semaphore()
        for nbr in (left, right):
            pl.semaphore_signal(barrier, inc=1, device_id=(nbr,),
                                device_id_type=pl.DeviceIdType.MESH)
        pl.semaphore_wait(barrier, 2)

        for d in range(2):
            comm_ref[d, 0, 0] = (
                k_ref[d].reshape(S, H * Dh).astype(jnp.bfloat16))
            comm_ref[d, 0, 1] = (
                v_ref[d].reshape(S, H * Dh).astype(jnp.bfloat16))

        xq = x_ref[...].reshape(B * S, D_MODEL).astype(jnp.bfloat16)
        wq = wq_ref[...].astype(jnp.bfloat16)
        q = jnp.dot(xq, wq, preferred_element_type=jnp.float32)
        q = (q * 0.125).astype(jnp.bfloat16)

        rows = lax.broadcasted_iota(jnp.int32, (S, S), 0)
        cols = lax.broadcasted_iota(jnp.int32, (S, S), 1)
        qb = my * (S // BLK) + rows // BLK

        den = [[None] * H for _ in range(B)]
        rdmas = [[], []]
        for h in range(N_DEV):
            if h > 0:
                rdmas[0][h - 1].wait()
                rdmas[1][h - 1].wait()
            if h < N_DEV - 1:
                for d, tgt in ((0, right), (1, left)):
                    r = pltpu.make_async_remote_copy(
                        src_ref=comm_ref.at[d, h],
                        dst_ref=comm_ref.at[d, h + 1],
                        send_sem=send_sems.at[d, h],
                        recv_sem=recv_sems.at[d, h],
                        device_id=(tgt,),
                        device_id_type=pl.DeviceIdType.MESH,
                    )
                    r.start()
                    rdmas[d].append(r)

            for d in range(2):
                b = d
                origin = lax.rem(my + (h if d else -h) + N_DEV, N_DEV)
                kb = origin * (S // BLK) + cols // BLK
                mask = (qb == kb) | (kb == 0) | (lax.rem(qb + kb, 3) == 0)
                for hd in range(H):
                    qbh = q[b * S:(b + 1) * S, hd * Dh:(hd + 1) * Dh]
                    kbh = comm_ref[d, h, 0, :, hd * Dh:(hd + 1) * Dh]
                    s = lax.dot_general(
                        qbh, kbh, (((1,), (1,)), ((), ())),
                        preferred_element_type=jnp.float32)
                    w = jnp.where(mask, jnp.exp(s), 0.0)
                    vbh = comm_ref[d, h, 1, :, hd * Dh:(hd + 1) * Dh]
                    pv = jnp.dot(w.astype(jnp.bfloat16), vbh,
                                 preferred_element_type=jnp.float32)
                    dsum = jnp.sum(w, axis=1, keepdims=True)
                    if h == 0:
                        acc_ref[pl.ds(b * S, S), pl.ds(hd * Dh, Dh)] = pv
                        den[b][hd] = dsum
                    else:
                        acc_ref[pl.ds(b * S, S), pl.ds(hd * Dh, Dh)] += pv
                        den[b][hd] = den[b][hd] + dsum

        for b in range(B):
            for hd in range(H):
                piece = acc_ref[pl.ds(b * S, S), pl.ds(hd * Dh, Dh)]
                ctx_ref[pl.ds(b * S, S), pl.ds(hd * Dh, Dh)] = (
                    piece / den[b][hd]).astype(jnp.bfloat16)

        out = jnp.dot(ctx_ref[...], wo_ref[...].astype(jnp.bfloat16),
                      preferred_element_type=jnp.float32)
        out_ref[...] = out.reshape(B, S, D_MODEL)

    return pl.pallas_call(
        body,
        out_shape=jax.ShapeDtypeStruct((B, S, D_MODEL), jnp.float32),
        in_specs=[pl.BlockSpec(memory_space=pltpu.VMEM)] * 5,
        out_specs=pl.BlockSpec(memory_space=pltpu.VMEM),
        scratch_shapes=[
            pltpu.VMEM((2, N_DEV, 2, S, H * Dh), jnp.bfloat16),
            pltpu.VMEM((B * S, H * Dh), jnp.float32),
            pltpu.VMEM((B * S, H * Dh), jnp.bfloat16),
            pltpu.SemaphoreType.DMA((2, N_DEV - 1)),
            pltpu.SemaphoreType.DMA((2, N_DEV - 1)),
        ],
        compiler_params=pltpu.CompilerParams(
            collective_id=0, vmem_limit_bytes=60 * 1024 * 1024),
    )(x, Wq, K_ext, V_ext, Wo)


# device time: 67725 ns/iter; 1.5197x vs baseline; 1.0135x over previous
import jax
import jax.numpy as jnp
from jax import lax
from jax.experimental import pallas as pl
from jax.experimental.pallas import tpu as pltpu

N_DEV = 4
B = 2
S = 512
H = 8
Dh = 64
D_MODEL = 768
BLK = 64


def kernel(x, Wq, K_ext, V_ext, Wo):
    def body(x_ref, wq_ref, k_ref, v_ref, wo_ref, out_ref,
             comm_ref, acc_ref, ctx_ref,
             send_sems, recv_sems):
        my = lax.axis_index("i")
        left = lax.rem(my + N_DEV - 1, N_DEV)
        right = lax.rem(my + 1, N_DEV)

        for d in range(2):
            comm_ref[d, 0, 0] = (
                k_ref[d].reshape(S, H * Dh).astype(jnp.bfloat16))
            comm_ref[d, 0, 1] = (
                v_ref[d].reshape(S, H * Dh).astype(jnp.bfloat16))

        barrier = pltpu.get_barrier_semaphore()
        for nbr in (left, right):
            pl.semaphore_signal(barrier, inc=1, device_id=(nbr,),
                                device_id_type=pl.DeviceIdType.MESH)
        pl.semaphore_wait(barrier, 2)

        rdmas = [[], []]
        def start_hop(h):
            for d, tgt in ((0, right), (1, left)):
                r = pltpu.make_async_remote_copy(
                    src_ref=comm_ref.at[d, h],
                    dst_ref=comm_ref.at[d, h + 1],
                    send_sem=send_sems.at[d, h],
                    recv_sem=recv_sems.at[d, h],
                    device_id=(tgt,),
                    device_id_type=pl.DeviceIdType.MESH,
                )
                r.start()
                rdmas[d].append(r)
        start_hop(0)

        xq = x_ref[...].reshape(B * S, D_MODEL).astype(jnp.bfloat16)
        wq = wq_ref[...].astype(jnp.bfloat16)
        q = jnp.dot(xq, wq, preferred_element_type=jnp.float32)
        q = (q * 0.125).astype(jnp.bfloat16)

        qblk = lax.broadcasted_iota(jnp.int32, (S, 1), 0) // BLK
        qb_g = my * (S // BLK) + qblk
        qr_need = lax.rem(3 - lax.rem(qb_g, 3), 3)
        cblk = lax.broadcasted_iota(jnp.int32, (1, S), 1) // BLK

        masks = {}
        def mask_for(k):
            if k not in masks:
                origin = lax.rem(my - k + N_DEV, N_DEV)
                kb_g = origin * (S // BLK) + cblk
                kr = lax.rem(kb_g, 3)
                masks[k] = (qb_g == kb_g) | (kb_g == 0) | (kr == qr_need)
            return masks[k]

        den = [[None] * H for _ in range(B)]
        for h in range(N_DEV):
            if h > 0:
                rdmas[0][h - 1].wait()
                rdmas[1][h - 1].wait()
                if h < N_DEV - 1:
                    start_hop(h)

            for d in range(2):
                b = d
                mask = mask_for(h if d == 0 else (N_DEV - h) % N_DEV)
                for hd in range(H):
                    qbh = q[b * S:(b + 1) * S, hd * Dh:(hd + 1) * Dh]
                    kbh = comm_ref[d, h, 0, :, hd * Dh:(hd + 1) * Dh]
                    s = lax.dot_general(
                        qbh, kbh, (((1,), (1,)), ((), ())),
                        preferred_element_type=jnp.float32)
                    w = jnp.where(mask, jnp.exp(s.astype(jnp.bfloat16)),
                                  jnp.bfloat16(0.0))
                    vbh = comm_ref[d, h, 1, :, hd * Dh:(hd + 1) * Dh]
                    pv = jnp.dot(w, vbh, preferred_element_type=jnp.float32)
                    dsum = jnp.sum(w.astype(jnp.float32), axis=1,
                                   keepdims=True)
                    if h == 0:
                        acc_ref[pl.ds(b * S, S), pl.ds(hd * Dh, Dh)] = pv
                        den[b][hd] = dsum
                    else:
                        acc_ref[pl.ds(b * S, S), pl.ds(hd * Dh, Dh)] += pv
                        den[b][hd] = den[b][hd] + dsum

        for b in range(B):
            for hd in range(H):
                piece = acc_ref[pl.ds(b * S, S), pl.ds(hd * Dh, Dh)]
                ctx_ref[pl.ds(b * S, S), pl.ds(hd * Dh, Dh)] = (
                    piece / den[b][hd]).astype(jnp.bfloat16)

        out = jnp.dot(ctx_ref[...], wo_ref[...].astype(jnp.bfloat16),
                      preferred_element_type=jnp.float32)
        out_ref[...] = out.reshape(B, S, D_MODEL)

    return pl.pallas_call(
        body,
        out_shape=jax.ShapeDtypeStruct((B, S, D_MODEL), jnp.float32),
        in_specs=[pl.BlockSpec(memory_space=pltpu.VMEM)] * 5,
        out_specs=pl.BlockSpec(memory_space=pltpu.VMEM),
        scratch_shapes=[
            pltpu.VMEM((2, N_DEV, 2, S, H * Dh), jnp.bfloat16),
            pltpu.VMEM((B * S, H * Dh), jnp.float32),
            pltpu.VMEM((B * S, H * Dh), jnp.bfloat16),
            pltpu.SemaphoreType.DMA((2, N_DEV - 1)),
            pltpu.SemaphoreType.DMA((2, N_DEV - 1)),
        ],
        compiler_params=pltpu.CompilerParams(
            collective_id=0, vmem_limit_bytes=60 * 1024 * 1024),
    )(x, Wq, K_ext, V_ext, Wo)


# device time: 61959 ns/iter; 1.6611x vs baseline; 1.0931x over previous
import jax
import jax.numpy as jnp
from jax import lax
from jax.experimental import pallas as pl
from jax.experimental.pallas import tpu as pltpu

N_DEV = 4
B = 2
S = 512
H = 8
Dh = 64
D_MODEL = 768
BLK = 64


def kernel(x, Wq, K_ext, V_ext, Wo):
    def body(x_ref, wq_ref, k_ref, v_ref, wo_ref, out_ref,
             comm_ref, acc_ref, ctx_ref, w_ref,
             send_sems, recv_sems):
        my = lax.axis_index("i")
        left = lax.rem(my + N_DEV - 1, N_DEV)
        right = lax.rem(my + 1, N_DEV)

        for d in range(2):
            comm_ref[d, 0, 0] = (
                k_ref[d].reshape(S, H * Dh).astype(jnp.bfloat16))
            comm_ref[d, 0, 1] = (
                v_ref[d].reshape(S, H * Dh).astype(jnp.bfloat16))

        barrier = pltpu.get_barrier_semaphore()
        for nbr in (left, right):
            pl.semaphore_signal(barrier, inc=1, device_id=(nbr,),
                                device_id_type=pl.DeviceIdType.MESH)
        pl.semaphore_wait(barrier, 2)

        k_rdmas = [[], []]
        v_rdmas = [[], []]

        def start_sub(h, kvi, rd):
            for d, tgt in ((0, right), (1, left)):
                r = pltpu.make_async_remote_copy(
                    src_ref=comm_ref.at[d, h, kvi],
                    dst_ref=comm_ref.at[d, h + 1, kvi],
                    send_sem=send_sems.at[d, kvi, h],
                    recv_sem=recv_sems.at[d, kvi, h],
                    device_id=(tgt,),
                    device_id_type=pl.DeviceIdType.MESH,
                )
                r.start()
                rd[d].append(r)

        start_sub(0, 0, k_rdmas)
        start_sub(0, 1, v_rdmas)

        xq = x_ref[...].reshape(B * S, D_MODEL).astype(jnp.bfloat16)
        wq = wq_ref[...].astype(jnp.bfloat16)
        q = jnp.dot(xq, wq, preferred_element_type=jnp.float32)
        q = (q * 0.125).astype(jnp.bfloat16)

        qblk = lax.broadcasted_iota(jnp.int32, (S, 1), 0) // BLK
        qb_g = my * (S // BLK) + qblk
        qr_need = lax.rem(3 - lax.rem(qb_g, 3), 3)
        cblk = lax.broadcasted_iota(jnp.int32, (1, S), 1) // BLK

        masks = {}

        def mask_for(k):
            if k not in masks:
                origin = lax.rem(my - k + N_DEV, N_DEV)
                kb_g = origin * (S // BLK) + cblk
                kr = lax.rem(kb_g, 3)
                masks[k] = (qb_g == kb_g) | (kb_g == 0) | (kr == qr_need)
            return masks[k]

        den = [[None] * H for _ in range(B)]
        for h in range(N_DEV):
            if h > 0:
                k_rdmas[0][h - 1].wait()
                k_rdmas[1][h - 1].wait()
                if h < N_DEV - 1:
                    start_sub(h, 0, k_rdmas)
            for d in range(2):
                b = d
                mask = mask_for(h if d == 0 else (N_DEV - h) % N_DEV)
                for hd in range(H):
                    qbh = q[b * S:(b + 1) * S, hd * Dh:(hd + 1) * Dh]
                    kbh = comm_ref[d, h, 0, :, hd * Dh:(hd + 1) * Dh]
                    s = lax.dot_general(
                        qbh, kbh, (((1,), (1,)), ((), ())),
                        preferred_element_type=jnp.float32)
                    w = jnp.where(mask, jnp.exp(s.astype(jnp.bfloat16)),
                                  jnp.bfloat16(0.0))
                    w_ref[d, :, pl.ds(hd * S, S)] = w
                    dsum = jnp.sum(w.astype(jnp.float32), axis=1,
                                   keepdims=True)
                    den[b][hd] = dsum if h == 0 else den[b][hd] + dsum

            if h > 0:
                v_rdmas[0][h - 1].wait()
                v_rdmas[1][h - 1].wait()
                if h < N_DEV - 1:
                    start_sub(h, 1, v_rdmas)
            for d in range(2):
                b = d
                for hd in range(H):
                    wv = w_ref[d, :, pl.ds(hd * S, S)]
                    vbh = comm_ref[d, h, 1, :, hd * Dh:(hd + 1) * Dh]
                    pv = jnp.dot(wv, vbh, preferred_element_type=jnp.float32)
                    if h == 0:
                        acc_ref[pl.ds(b * S, S), pl.ds(hd * Dh, Dh)] = pv
                    else:
                        acc_ref[pl.ds(b * S, S), pl.ds(hd * Dh, Dh)] += pv

        for b in range(B):
            for hd in range(H):
                piece = acc_ref[pl.ds(b * S, S), pl.ds(hd * Dh, Dh)]
                ctx_ref[pl.ds(b * S, S), pl.ds(hd * Dh, Dh)] = (
                    piece / den[b][hd]).astype(jnp.bfloat16)

        out = jnp.dot(ctx_ref[...], wo_ref[...].astype(jnp.bfloat16),
                      preferred_element_type=jnp.float32)
        out_ref[...] = out.reshape(B, S, D_MODEL)

    return pl.pallas_call(
        body,
        out_shape=jax.ShapeDtypeStruct((B, S, D_MODEL), jnp.float32),
        in_specs=[pl.BlockSpec(memory_space=pltpu.VMEM)] * 5,
        out_specs=pl.BlockSpec(memory_space=pltpu.VMEM),
        scratch_shapes=[
            pltpu.VMEM((2, N_DEV, 2, S, H * Dh), jnp.bfloat16),
            pltpu.VMEM((B * S, H * Dh), jnp.float32),
            pltpu.VMEM((B * S, H * Dh), jnp.bfloat16),
            pltpu.VMEM((2, S, H * S), jnp.bfloat16),
            pltpu.SemaphoreType.DMA((2, 2, N_DEV - 1)),
            pltpu.SemaphoreType.DMA((2, 2, N_DEV - 1)),
        ],
        compiler_params=pltpu.CompilerParams(
            collective_id=0, vmem_limit_bytes=60 * 1024 * 1024),
    )(x, Wq, K_ext, V_ext, Wo)


# device time: 48853 ns/iter; 2.1068x vs baseline; 1.2683x over previous
import jax
import jax.numpy as jnp
from jax import lax
from jax.experimental import pallas as pl
from jax.experimental.pallas import tpu as pltpu

N_DEV = 4
B = 2
S = 512
H = 8
Dh = 64
D_MODEL = 768
BLK = 64


def kernel(x, Wq, K_ext, V_ext, Wo):
    def body(x_ref, wq_ref, k_ref, v_ref, wo_ref, out_ref,
             comm_ref, acc_ref, ctx_ref, w_ref,
             send_sems, recv_sems):
        my = lax.axis_index("i")
        left = lax.rem(my + N_DEV - 1, N_DEV)
        right = lax.rem(my + 1, N_DEV)

        for d in range(2):
            comm_ref[d, 0, 0] = (
                k_ref[d].reshape(S, H * Dh).astype(jnp.float8_e4m3fn))
            comm_ref[d, 0, 1] = (
                v_ref[d].reshape(S, H * Dh).astype(jnp.float8_e4m3fn))

        barrier = pltpu.get_barrier_semaphore()
        for nbr in (left, right):
            pl.semaphore_signal(barrier, inc=1, device_id=(nbr,),
                                device_id_type=pl.DeviceIdType.MESH)
        pl.semaphore_wait(barrier, 2)

        k_rdmas = [[], []]
        v_rdmas = [[], []]

        def start_sub(h, kvi, rd):
            for d, tgt in ((0, right), (1, left)):
                r = pltpu.make_async_remote_copy(
                    src_ref=comm_ref.at[d, h, kvi],
                    dst_ref=comm_ref.at[d, h + 1, kvi],
                    send_sem=send_sems.at[d, kvi, h],
                    recv_sem=recv_sems.at[d, kvi, h],
                    device_id=(tgt,),
                    device_id_type=pl.DeviceIdType.MESH,
                )
                r.start()
                rd[d].append(r)

        start_sub(0, 0, k_rdmas)
        start_sub(0, 1, v_rdmas)

        xq = x_ref[...].reshape(B * S, D_MODEL).astype(jnp.bfloat16)
        wq = wq_ref[...].astype(jnp.bfloat16)
        q = jnp.dot(xq, wq, preferred_element_type=jnp.float32)
        q = (q * 0.125).astype(jnp.bfloat16)

        qblk = lax.broadcasted_iota(jnp.int32, (S, 1), 0) // BLK
        qb_g = my * (S // BLK) + qblk
        qr_need = lax.rem(3 - lax.rem(qb_g, 3), 3)
        cblk = lax.broadcasted_iota(jnp.int32, (1, S), 1) // BLK

        masks = {}

        def mask_for(k):
            if k not in masks:
                origin = lax.rem(my - k + N_DEV, N_DEV)
                kb_g = origin * (S // BLK) + cblk
                kr = lax.rem(kb_g, 3)
                masks[k] = (qb_g == kb_g) | (kb_g == 0) | (kr == qr_need)
            return masks[k]

        den = [[None] * H for _ in range(B)]
        for h in range(N_DEV):
            if h > 0:
                k_rdmas[0][h - 1].wait()
                k_rdmas[1][h - 1].wait()
                if h < N_DEV - 1:
                    start_sub(h, 0, k_rdmas)
            for d in range(2):
                b = d
                mask = mask_for(h if d == 0 else (N_DEV - h) % N_DEV)
                for hd in range(H):
                    qbh = q[b * S:(b + 1) * S, hd * Dh:(hd + 1) * Dh]
                    kbh = comm_ref[d, h, 0, :, hd * Dh:(hd + 1) * Dh].astype(
                        jnp.bfloat16)
                    s = lax.dot_general(
                        qbh, kbh, (((1,), (1,)), ((), ())),
                        preferred_element_type=jnp.float32)
                    w = jnp.where(mask, jnp.exp(s.astype(jnp.bfloat16)),
                                  jnp.bfloat16(0.0))
                    w_ref[d, :, pl.ds(hd * S, S)] = w
                    dsum = jnp.sum(w.astype(jnp.float32), axis=1,
                                   keepdims=True)
                    den[b][hd] = dsum if h == 0 else den[b][hd] + dsum

            if h > 0:
                v_rdmas[0][h - 1].wait()
                v_rdmas[1][h - 1].wait()
                if h < N_DEV - 1:
                    start_sub(h, 1, v_rdmas)
            for d in range(2):
                b = d
                for hd in range(H):
                    wv = w_ref[d, :, pl.ds(hd * S, S)]
                    vbh = comm_ref[d, h, 1, :, hd * Dh:(hd + 1) * Dh].astype(
                        jnp.bfloat16)
                    pv = jnp.dot(wv, vbh, preferred_element_type=jnp.float32)
                    if h == 0:
                        acc_ref[pl.ds(b * S, S), pl.ds(hd * Dh, Dh)] = pv
                    else:
                        acc_ref[pl.ds(b * S, S), pl.ds(hd * Dh, Dh)] += pv

        for b in range(B):
            for hd in range(H):
                piece = acc_ref[pl.ds(b * S, S), pl.ds(hd * Dh, Dh)]
                ctx_ref[pl.ds(b * S, S), pl.ds(hd * Dh, Dh)] = (
                    piece / den[b][hd]).astype(jnp.bfloat16)

        out = jnp.dot(ctx_ref[...], wo_ref[...].astype(jnp.bfloat16),
                      preferred_element_type=jnp.float32)
        out_ref[...] = out.reshape(B, S, D_MODEL)

    return pl.pallas_call(
        body,
        out_shape=jax.ShapeDtypeStruct((B, S, D_MODEL), jnp.float32),
        in_specs=[pl.BlockSpec(memory_space=pltpu.VMEM)] * 5,
        out_specs=pl.BlockSpec(memory_space=pltpu.VMEM),
        scratch_shapes=[
            pltpu.VMEM((2, N_DEV, 2, S, H * Dh), jnp.float8_e4m3fn),
            pltpu.VMEM((B * S, H * Dh), jnp.float32),
            pltpu.VMEM((B * S, H * Dh), jnp.bfloat16),
            pltpu.VMEM((2, S, H * S), jnp.bfloat16),
            pltpu.SemaphoreType.DMA((2, 2, N_DEV - 1)),
            pltpu.SemaphoreType.DMA((2, 2, N_DEV - 1)),
        ],
        compiler_params=pltpu.CompilerParams(
            collective_id=0, vmem_limit_bytes=60 * 1024 * 1024),
    )(x, Wq, K_ext, V_ext, Wo)
